# Initial kernel scaffold; baseline (speedup 1.0000x reference)
import jax
import jax.numpy as jnp
from jax.experimental import pallas as pl


def kernel(queries, keys, values, attn_mask):
    raise NotImplementedError("write your pallas kernel here")


# speedup vs baseline: 20.6189x; 20.6189x over previous
import functools

import jax
import jax.numpy as jnp
import numpy as np
from jax.experimental import pallas as pl
from jax.experimental.pallas import tpu as pltpu

_NB = 8


def _dotT(a, x):
    return jax.lax.dot_general(
        a, x, (((0,), (0,)), ((), ())), preferred_element_type=jnp.float32
    )


def _q_kernel(q_ref, cs_ref, qq_ref, acc_ref, *, n_nodes, nb):
    n = pl.program_id(1)
    x = q_ref[0, 0]
    for i in range(1, nb):
        x = x + q_ref[0, i]

    @pl.when(n == 0)
    def _():
        acc_ref[...] = x

    @pl.when(n > 0)
    def _():
        acc_ref[...] += x

    @pl.when(n == n_nodes // nb - 1)
    def _():
        qm = acc_ref[...] * (1.0 / n_nodes)
        qq_ref[0] = _dotT(cs_ref[...], qm)


def _main_kernel(
    k_ref, v_ref, qq_ref, cs_ref, cisi_ref, r_ref, rb_ref, out_ref, *, length, nb
):
    cs = cs_ref[...]
    cisi = cisi_ref[...]
    r = r_ref[...]
    rb = rb_ref[...]
    qq = qq_ref[0]
    qc = qq[:length]
    qs = qq[length:]
    for i in range(nb):
        kk = _dotT(cs, k_ref[0, i])
        kc = kk[:length]
        ks = kk[length:]
        pre = qc * kc + qs * ks
        pim = qc * ks - qs * kc
        p2 = jnp.concatenate([pre, pim], axis=0)
        x = jnp.dot(p2, r, preferred_element_type=jnp.float32)
        corr = _dotT(cisi, x)
        m1 = jnp.max(corr, axis=0, keepdims=True)
        d_iota = jax.lax.broadcasted_iota(jnp.int32, corr.shape, 0)
        i1 = jnp.min(jnp.where(corr == m1, d_iota, length), axis=0, keepdims=True)
        m2 = jnp.max(jnp.where(d_iota == i1, -jnp.inf, corr), axis=0, keepdims=True)
        scale = (jax.nn.sigmoid(m1) + jax.nn.sigmoid(m2)) * 0.5
        srow = jnp.dot(scale, rb, preferred_element_type=jnp.float32)
        out_ref[0, i] = v_ref[0, i] * srow


def kernel(queries, keys, values, attn_mask):
    B, N, L, H, E = queries.shape
    HE = H * E
    NB = _NB
    q4 = queries.reshape(B, N, L, HE)
    k4 = keys.reshape(B, N, L, HE)
    v4 = values.reshape(B, N, L, HE)

    t = np.arange(L)
    ang = 2.0 * np.pi * np.outer(t, t) / L
    Cnp = np.cos(ang).astype(np.float32)
    Snp = np.sin(ang).astype(np.float32)
    CS = jnp.asarray(np.concatenate([Cnp, Snp], axis=1))
    CiSi = jnp.asarray(
        np.concatenate([Cnp, -Snp], axis=0) * (1.0 / (L * E))
    )
    he = np.arange(HE)
    Rnp = np.zeros((HE, L), dtype=np.float32)
    Rnp[he, he // E] = 1.0
    R = jnp.asarray(Rnp)
    Rbnp = np.zeros((L, HE), dtype=np.float32)
    Rbnp[he // E, he] = 1.0
    Rb = jnp.asarray(Rbnp)

    def full(shape):
        return pl.BlockSpec(shape, lambda b, n: (0,) * len(shape))

    blk4 = pl.BlockSpec((1, NB, L, HE), lambda b, n: (b, n, 0, 0))
    blkq = pl.BlockSpec((1, 2 * L, HE), lambda b, n: (b, 0, 0))

    qq = pl.pallas_call(
        functools.partial(_q_kernel, n_nodes=N, nb=NB),
        grid=(B, N // NB),
        in_specs=[blk4, full((L, 2 * L))],
        out_specs=blkq,
        out_shape=jax.ShapeDtypeStruct((B, 2 * L, HE), jnp.float32),
        scratch_shapes=[pltpu.VMEM((L, HE), jnp.float32)],
        compiler_params=pltpu.CompilerParams(
            dimension_semantics=("parallel", "arbitrary")
        ),
    )(q4, CS)

    out4 = pl.pallas_call(
        functools.partial(_main_kernel, length=L, nb=NB),
        grid=(B, N // NB),
        in_specs=[
            blk4,
            blk4,
            blkq,
            full((L, 2 * L)),
            full((2 * L, L)),
            full((HE, L)),
            full((L, HE)),
        ],
        out_specs=blk4,
        out_shape=jax.ShapeDtypeStruct((B, N, L, HE), jnp.float32),
        compiler_params=pltpu.CompilerParams(
            dimension_semantics=("parallel", "parallel")
        ),
    )(k4, v4, qq, CS, CiSi, R, Rb)

    return out4.reshape(B, N, L, H, E)


# speedup vs baseline: 20.6232x; 1.0002x over previous
import functools

import jax
import jax.numpy as jnp
import numpy as np
from jax.experimental import pallas as pl
from jax.experimental.pallas import tpu as pltpu

_NB = 8


def _dotT(a, x):
    return jax.lax.dot_general(
        a, x, (((0,), (0,)), ((), ())), preferred_element_type=jnp.float32
    )


def _fused_kernel(
    q_ref,
    k_ref,
    v_ref,
    cs_ref,
    cisi_ref,
    r_ref,
    rb_ref,
    out_ref,
    acc_ref,
    qq_ref,
    *,
    n_nodes,
    nb,
    length,
):
    p = pl.program_id(1)
    n = pl.program_id(2)

    @pl.when(p == 0)
    def _():
        x = q_ref[0, 0]
        for i in range(1, nb):
            x = x + q_ref[0, i]

        @pl.when(n == 0)
        def _():
            acc_ref[...] = x

        @pl.when(n > 0)
        def _():
            acc_ref[...] += x

        @pl.when(n == n_nodes // nb - 1)
        def _():
            qm = acc_ref[...] * (1.0 / n_nodes)
            qq_ref[...] = _dotT(cs_ref[...], qm)

    @pl.when(p == 1)
    def _():
        cs = cs_ref[...]
        cisi = cisi_ref[...]
        r = r_ref[...]
        rb = rb_ref[...]
        qq = qq_ref[...]
        qc = qq[:length]
        qs = qq[length:]
        for i in range(nb):
            kk = _dotT(cs, k_ref[0, i])
            kc = kk[:length]
            ks = kk[length:]
            pre = qc * kc + qs * ks
            pim = qc * ks - qs * kc
            p2 = jnp.concatenate([pre, pim], axis=0)
            x = jnp.dot(p2, r, preferred_element_type=jnp.float32)
            corr = _dotT(cisi, x)
            m1 = jnp.max(corr, axis=0, keepdims=True)
            d_iota = jax.lax.broadcasted_iota(jnp.int32, corr.shape, 0)
            i1 = jnp.min(
                jnp.where(corr == m1, d_iota, length), axis=0, keepdims=True
            )
            m2 = jnp.max(
                jnp.where(d_iota == i1, -jnp.inf, corr), axis=0, keepdims=True
            )
            scale = (jax.nn.sigmoid(m1) + jax.nn.sigmoid(m2)) * 0.5
            srow = jnp.dot(scale, rb, preferred_element_type=jnp.float32)
            out_ref[0, i] = v_ref[0, i] * srow


def kernel(queries, keys, values, attn_mask):
    B, N, L, H, E = queries.shape
    HE = H * E
    NB = _NB
    NSTEP = N // NB
    q4 = queries.reshape(B, N, L, HE)
    k4 = keys.reshape(B, N, L, HE)
    v4 = values.reshape(B, N, L, HE)

    t = np.arange(L)
    ang = 2.0 * np.pi * np.outer(t, t) / L
    Cnp = np.cos(ang).astype(np.float32)
    Snp = np.sin(ang).astype(np.float32)
    CS = jnp.asarray(np.concatenate([Cnp, Snp], axis=1))
    CiSi = jnp.asarray(
        np.concatenate([Cnp, -Snp], axis=0) * (1.0 / (L * E))
    )
    he = np.arange(HE)
    Rnp = np.zeros((HE, L), dtype=np.float32)
    Rnp[he, he // E] = 1.0
    R = jnp.asarray(Rnp)
    Rbnp = np.zeros((L, HE), dtype=np.float32)
    Rbnp[he // E, he] = 1.0
    Rb = jnp.asarray(Rbnp)

    def full(shape):
        return pl.BlockSpec(shape, lambda b, p, n: (0,) * len(shape))

    q_spec = pl.BlockSpec(
        (1, NB, L, HE),
        lambda b, p, n: (b, jnp.where(p == 0, n, NSTEP - 1), 0, 0),
    )
    kv_spec = pl.BlockSpec(
        (1, NB, L, HE),
        lambda b, p, n: (b, jnp.where(p == 1, n, 0), 0, 0),
    )

    out4 = pl.pallas_call(
        functools.partial(_fused_kernel, n_nodes=N, nb=NB, length=L),
        grid=(B, 2, NSTEP),
        in_specs=[
            q_spec,
            kv_spec,
            kv_spec,
            full((L, 2 * L)),
            full((2 * L, L)),
            full((HE, L)),
            full((L, HE)),
        ],
        out_specs=kv_spec,
        out_shape=jax.ShapeDtypeStruct((B, N, L, HE), jnp.float32),
        scratch_shapes=[
            pltpu.VMEM((L, HE), jnp.float32),
            pltpu.VMEM((2 * L, HE), jnp.float32),
        ],
        compiler_params=pltpu.CompilerParams(
            dimension_semantics=("arbitrary", "arbitrary", "arbitrary")
        ),
    )(q4, k4, v4, CS, CiSi, R, Rb)

    return out4.reshape(B, N, L, H, E)
